# baseline (speedup 1.0000x reference)
import functools

import jax
import jax.numpy as jnp
from jax import lax
from jax.experimental import pallas as pl
from jax.experimental.pallas import tpu as pltpu
from jax.experimental.pallas import tpu_sc as plsc

_EMBED_DIM = 64
_EMBEDS = 1024
_ROWS = 16384
_BLK_R = 512
_NB = _ROWS // _BLK_R

_NC = 2
_NS = 16
_NW = _NC * _NS
_B_PER_W = _ROWS // _NW


def _argmin_loss_body(flat_ref, emb_ref, idx_ref, loss_ref):
    i = pl.program_id(0)
    blk = flat_ref[...]
    emb = emb_ref[...]
    row_sq = jnp.sum(blk * blk, axis=1, keepdims=True)
    emb_sq = jnp.sum(emb * emb, axis=0, keepdims=True)
    half = _EMBEDS // 4
    m = None
    idxf = None
    for c in range(4):
        emb_c = emb[:, c * half:(c + 1) * half]
        prod_c = jnp.dot(blk, emb_c, preferred_element_type=jnp.float32)
        d_c = row_sq + emb_sq[:, c * half:(c + 1) * half] - 2.0 * prod_c
        m_c = jnp.min(d_c, axis=1, keepdims=True)
        iot = lax.broadcasted_iota(jnp.int32, d_c.shape, 1).astype(jnp.float32)
        i_c = jnp.min(jnp.where(d_c == m_c, iot, float(_EMBEDS)), axis=1)
        i_c = i_c + float(c * half)
        if m is None:
            m, idxf = m_c, i_c
        else:
            idxf = jnp.where(m_c[:, 0] < m[:, 0], i_c, idxf)
            m = jnp.minimum(m, m_c)
    idx_ref[...] = idxf.astype(jnp.int32)

    @pl.when(i == 0)
    def _init():
        loss_ref[0, 0] = 0.0

    loss_ref[0, 0] += jnp.sum(m)

    @pl.when(i == pl.num_programs(0) - 1)
    def _finish():
        loss_ref[0, 0] = loss_ref[0, 0] * (1.25 / (_ROWS * _EMBED_DIM))


def _argmin_loss(flat, embeddings, rows, off):
    nb = rows // _BLK_R
    off_nb = off // _BLK_R
    return pl.pallas_call(
        _argmin_loss_body,
        grid=(nb,),
        in_specs=[
            pl.BlockSpec((_BLK_R, _EMBED_DIM), lambda i: (i + off_nb, 0)),
            pl.BlockSpec((_EMBED_DIM, _EMBEDS), lambda i: (0, 0)),
        ],
        out_specs=[
            pl.BlockSpec((_BLK_R,), lambda i: (i,)),
            pl.BlockSpec(memory_space=pltpu.SMEM),
        ],
        out_shape=[
            jax.ShapeDtypeStruct((rows,), jnp.int32),
            jax.ShapeDtypeStruct((1, 1), jnp.float32),
        ],
        compiler_params=pltpu.CompilerParams(
            dimension_semantics=("arbitrary",),
        ),
    )(flat, embeddings)


def _sc_gather(table, idx, rows):
    b_per_w = rows // _NW
    mesh = plsc.VectorSubcoreMesh(
        core_axis_name="c", subcore_axis_name="s",
        num_cores=_NC, num_subcores=_NS,
    )

    @functools.partial(
        pl.kernel,
        mesh=mesh,
        out_type=jax.ShapeDtypeStruct((rows // 1024, 32, 32, _EMBED_DIM), jnp.float32),
        scratch_types=[
            pltpu.VMEM((b_per_w,), jnp.int32),
            pltpu.VMEM((b_per_w, _EMBED_DIM), jnp.float32),
            pltpu.SemaphoreType.DMA,
        ],
        compiler_params=pltpu.CompilerParams(use_tc_tiling_on_sc=False),
    )
    def gather_k(table_hbm, idx_hbm, out_hbm, idx_v, rows_v, sem):
        wid = lax.axis_index("s") * _NC + lax.axis_index("c")
        base = wid * b_per_w
        pltpu.sync_copy(idx_hbm.at[pl.ds(base, b_per_w)], idx_v)
        pltpu.async_copy(table_hbm.at[idx_v], rows_v, sem).wait()
        img = base // 1024
        i0 = (base % 1024) // 32
        handles = [
            pltpu.async_copy(
                rows_v.at[pl.ds(j * 32, 32)], out_hbm.at[img, i0 + j], sem)
            for j in range(b_per_w // 32)
        ]
        for h in handles:
            h.wait()

    return gather_k(table, idx)


def kernel(x, embeddings):
    in_shape = x.shape
    flat = x.reshape(-1, _EMBED_DIM)
    table = embeddings.T
    half = _ROWS // 2
    idx, l0 = _argmin_loss(flat, embeddings, _ROWS, 0)
    qtised = _sc_gather(table, idx, _ROWS)
    return qtised, l0[0, 0]


# speedup vs baseline: 1.0986x; 1.0986x over previous
import functools

import jax
import jax.numpy as jnp
from jax import lax
from jax.experimental import pallas as pl
from jax.experimental.pallas import tpu as pltpu
from jax.experimental.pallas import tpu_sc as plsc

_EMBED_DIM = 64
_EMBEDS = 1024
_ROWS = 16384
_BLK_R = 1024
_NB = _ROWS // _BLK_R

_NC = 2
_NS = 16
_NW = _NC * _NS
_B_PER_W = _ROWS // _NW


def _argmin_loss_body(flat_ref, emb_ref, idx_ref, loss_ref):
    i = pl.program_id(0)
    blk = flat_ref[...]
    emb = emb_ref[...]
    row_sq = jnp.sum(blk * blk, axis=1, keepdims=True)
    emb_sq = jnp.sum(emb * emb, axis=0, keepdims=True)
    half = _EMBEDS // 2
    m = None
    idxf = None
    for c in range(2):
        emb_c = emb[:, c * half:(c + 1) * half]
        prod_c = jnp.dot(blk, emb_c, preferred_element_type=jnp.float32)
        d_c = row_sq + emb_sq[:, c * half:(c + 1) * half] - 2.0 * prod_c
        m_c = jnp.min(d_c, axis=1, keepdims=True)
        iot = lax.broadcasted_iota(jnp.int32, d_c.shape, 1).astype(jnp.float32)
        i_c = jnp.min(jnp.where(d_c == m_c, iot, float(_EMBEDS)), axis=1)
        i_c = i_c + float(c * half)
        if m is None:
            m, idxf = m_c, i_c
        else:
            idxf = jnp.where(m_c[:, 0] < m[:, 0], i_c, idxf)
            m = jnp.minimum(m, m_c)
    idx_ref[...] = idxf.astype(jnp.int32)

    @pl.when(i == 0)
    def _init():
        loss_ref[0, 0] = 0.0

    loss_ref[0, 0] += jnp.sum(m)

    @pl.when(i == pl.num_programs(0) - 1)
    def _finish():
        loss_ref[0, 0] = loss_ref[0, 0] * (1.25 / (_ROWS * _EMBED_DIM))


def _argmin_loss(flat, embeddings, rows, off):
    nb = rows // _BLK_R
    off_nb = off // _BLK_R
    return pl.pallas_call(
        _argmin_loss_body,
        grid=(nb,),
        in_specs=[
            pl.BlockSpec((_BLK_R, _EMBED_DIM), lambda i: (i + off_nb, 0)),
            pl.BlockSpec((_EMBED_DIM, _EMBEDS), lambda i: (0, 0)),
        ],
        out_specs=[
            pl.BlockSpec((_BLK_R,), lambda i: (i,)),
            pl.BlockSpec(memory_space=pltpu.SMEM),
        ],
        out_shape=[
            jax.ShapeDtypeStruct((rows,), jnp.int32),
            jax.ShapeDtypeStruct((1, 1), jnp.float32),
        ],
        compiler_params=pltpu.CompilerParams(
            dimension_semantics=("arbitrary",),
        ),
    )(flat, embeddings)


def _sc_gather(table, idx, rows):
    b_per_w = rows // _NW
    mesh = plsc.VectorSubcoreMesh(
        core_axis_name="c", subcore_axis_name="s",
        num_cores=_NC, num_subcores=_NS,
    )

    @functools.partial(
        pl.kernel,
        mesh=mesh,
        out_type=jax.ShapeDtypeStruct((rows // 1024, 32, 32, _EMBED_DIM), jnp.float32),
        scratch_types=[
            pltpu.VMEM((b_per_w,), jnp.int32),
            pltpu.VMEM((b_per_w, _EMBED_DIM), jnp.float32),
            pltpu.SemaphoreType.DMA,
        ],
        compiler_params=pltpu.CompilerParams(use_tc_tiling_on_sc=False),
    )
    def gather_k(table_hbm, idx_hbm, out_hbm, idx_v, rows_v, sem):
        wid = lax.axis_index("s") * _NC + lax.axis_index("c")
        base = wid * b_per_w
        pltpu.sync_copy(idx_hbm.at[pl.ds(base, b_per_w)], idx_v)
        pltpu.async_copy(table_hbm.at[idx_v], rows_v, sem).wait()
        img = base // 1024
        i0 = (base % 1024) // 32
        handles = [
            pltpu.async_copy(
                rows_v.at[pl.ds(j * 32, 32)], out_hbm.at[img, i0 + j], sem)
            for j in range(b_per_w // 32)
        ]
        for h in handles:
            h.wait()

    return gather_k(table, idx)


def kernel(x, embeddings):
    in_shape = x.shape
    flat = x.reshape(-1, _EMBED_DIM)
    table = embeddings.T
    half = _ROWS // 2
    idx, l0 = _argmin_loss(flat, embeddings, _ROWS, 0)
    qtised = _sc_gather(table, idx, _ROWS)
    return qtised, l0[0, 0]


# speedup vs baseline: 1.1861x; 1.0796x over previous
import functools

import jax
import jax.numpy as jnp
from jax import lax
from jax.experimental import pallas as pl
from jax.experimental.pallas import tpu as pltpu
from jax.experimental.pallas import tpu_sc as plsc

_EMBED_DIM = 64
_EMBEDS = 1024
_ROWS = 16384
_BLK_R = 2048
_NB = _ROWS // _BLK_R

_NC = 2
_NS = 16
_NW = _NC * _NS
_B_PER_W = _ROWS // _NW


def _argmin_loss_body(flat_ref, emb_ref, idx_ref, loss_ref):
    i = pl.program_id(0)
    blk = flat_ref[...]
    emb = emb_ref[...]
    row_sq = jnp.sum(blk * blk, axis=1, keepdims=True)
    emb_sq = jnp.sum(emb * emb, axis=0, keepdims=True)
    half = _EMBEDS // 2
    m = None
    idxf = None
    for c in range(2):
        emb_c = emb[:, c * half:(c + 1) * half]
        prod_c = jnp.dot(blk, emb_c, preferred_element_type=jnp.float32)
        d_c = row_sq + emb_sq[:, c * half:(c + 1) * half] - 2.0 * prod_c
        m_c = jnp.min(d_c, axis=1, keepdims=True)
        iot = lax.broadcasted_iota(jnp.int32, d_c.shape, 1).astype(jnp.float32)
        i_c = jnp.min(jnp.where(d_c == m_c, iot, float(_EMBEDS)), axis=1)
        i_c = i_c + float(c * half)
        if m is None:
            m, idxf = m_c, i_c
        else:
            idxf = jnp.where(m_c[:, 0] < m[:, 0], i_c, idxf)
            m = jnp.minimum(m, m_c)
    idx_ref[...] = idxf.astype(jnp.int32)

    @pl.when(i == 0)
    def _init():
        loss_ref[0, 0] = 0.0

    loss_ref[0, 0] += jnp.sum(m)

    @pl.when(i == pl.num_programs(0) - 1)
    def _finish():
        loss_ref[0, 0] = loss_ref[0, 0] * (1.25 / (_ROWS * _EMBED_DIM))


def _argmin_loss(flat, embeddings, rows, off):
    nb = rows // _BLK_R
    off_nb = off // _BLK_R
    return pl.pallas_call(
        _argmin_loss_body,
        grid=(nb,),
        in_specs=[
            pl.BlockSpec((_BLK_R, _EMBED_DIM), lambda i: (i + off_nb, 0)),
            pl.BlockSpec((_EMBED_DIM, _EMBEDS), lambda i: (0, 0)),
        ],
        out_specs=[
            pl.BlockSpec((_BLK_R,), lambda i: (i,)),
            pl.BlockSpec(memory_space=pltpu.SMEM),
        ],
        out_shape=[
            jax.ShapeDtypeStruct((rows,), jnp.int32),
            jax.ShapeDtypeStruct((1, 1), jnp.float32),
        ],
        compiler_params=pltpu.CompilerParams(
            dimension_semantics=("arbitrary",),
        ),
    )(flat, embeddings)


def _sc_gather(table, idx, rows):
    b_per_w = rows // _NW
    mesh = plsc.VectorSubcoreMesh(
        core_axis_name="c", subcore_axis_name="s",
        num_cores=_NC, num_subcores=_NS,
    )

    @functools.partial(
        pl.kernel,
        mesh=mesh,
        out_type=jax.ShapeDtypeStruct((rows // 1024, 32, 32, _EMBED_DIM), jnp.float32),
        scratch_types=[
            pltpu.VMEM((b_per_w,), jnp.int32),
            pltpu.VMEM((b_per_w, _EMBED_DIM), jnp.float32),
            pltpu.SemaphoreType.DMA,
        ],
        compiler_params=pltpu.CompilerParams(use_tc_tiling_on_sc=False),
    )
    def gather_k(table_hbm, idx_hbm, out_hbm, idx_v, rows_v, sem):
        wid = lax.axis_index("s") * _NC + lax.axis_index("c")
        base = wid * b_per_w
        pltpu.sync_copy(idx_hbm.at[pl.ds(base, b_per_w)], idx_v)
        pltpu.async_copy(table_hbm.at[idx_v], rows_v, sem).wait()
        img = base // 1024
        i0 = (base % 1024) // 32
        handles = [
            pltpu.async_copy(
                rows_v.at[pl.ds(j * 32, 32)], out_hbm.at[img, i0 + j], sem)
            for j in range(b_per_w // 32)
        ]
        for h in handles:
            h.wait()

    return gather_k(table, idx)


def kernel(x, embeddings):
    in_shape = x.shape
    flat = x.reshape(-1, _EMBED_DIM)
    table = embeddings.T
    half = _ROWS // 2
    idx, l0 = _argmin_loss(flat, embeddings, _ROWS, 0)
    qtised = _sc_gather(table, idx, _ROWS)
    return qtised, l0[0, 0]


# speedup vs baseline: 1.1971x; 1.0093x over previous
import functools

import jax
import jax.numpy as jnp
from jax import lax
from jax.experimental import pallas as pl
from jax.experimental.pallas import tpu as pltpu
from jax.experimental.pallas import tpu_sc as plsc

_EMBED_DIM = 64
_EMBEDS = 1024
_ROWS = 16384
_BLK_R = 4096
_NB = _ROWS // _BLK_R

_NC = 2
_NS = 16
_NW = _NC * _NS
_B_PER_W = _ROWS // _NW


def _argmin_loss_body(flat_ref, emb_ref, idx_ref, loss_ref):
    i = pl.program_id(0)
    blk = flat_ref[...]
    emb = emb_ref[...]
    row_sq = jnp.sum(blk * blk, axis=1, keepdims=True)
    emb_sq = jnp.sum(emb * emb, axis=0, keepdims=True)
    half = _EMBEDS // 2
    m = None
    idxf = None
    for c in range(2):
        emb_c = emb[:, c * half:(c + 1) * half]
        prod_c = jnp.dot(blk, emb_c, preferred_element_type=jnp.float32)
        d_c = row_sq + emb_sq[:, c * half:(c + 1) * half] - 2.0 * prod_c
        m_c = jnp.min(d_c, axis=1, keepdims=True)
        iot = lax.broadcasted_iota(jnp.int32, d_c.shape, 1).astype(jnp.float32)
        i_c = jnp.min(jnp.where(d_c == m_c, iot, float(_EMBEDS)), axis=1)
        i_c = i_c + float(c * half)
        if m is None:
            m, idxf = m_c, i_c
        else:
            idxf = jnp.where(m_c[:, 0] < m[:, 0], i_c, idxf)
            m = jnp.minimum(m, m_c)
    idx_ref[...] = idxf.astype(jnp.int32)

    @pl.when(i == 0)
    def _init():
        loss_ref[0, 0] = 0.0

    loss_ref[0, 0] += jnp.sum(m)

    @pl.when(i == pl.num_programs(0) - 1)
    def _finish():
        loss_ref[0, 0] = loss_ref[0, 0] * (1.25 / (_ROWS * _EMBED_DIM))


def _argmin_loss(flat, embeddings, rows, off):
    nb = rows // _BLK_R
    off_nb = off // _BLK_R
    return pl.pallas_call(
        _argmin_loss_body,
        grid=(nb,),
        in_specs=[
            pl.BlockSpec((_BLK_R, _EMBED_DIM), lambda i: (i + off_nb, 0)),
            pl.BlockSpec((_EMBED_DIM, _EMBEDS), lambda i: (0, 0)),
        ],
        out_specs=[
            pl.BlockSpec((_BLK_R,), lambda i: (i,)),
            pl.BlockSpec(memory_space=pltpu.SMEM),
        ],
        out_shape=[
            jax.ShapeDtypeStruct((rows,), jnp.int32),
            jax.ShapeDtypeStruct((1, 1), jnp.float32),
        ],
        compiler_params=pltpu.CompilerParams(
            dimension_semantics=("arbitrary",),
        ),
    )(flat, embeddings)


def _sc_gather(table, idx, rows):
    b_per_w = rows // _NW
    mesh = plsc.VectorSubcoreMesh(
        core_axis_name="c", subcore_axis_name="s",
        num_cores=_NC, num_subcores=_NS,
    )

    @functools.partial(
        pl.kernel,
        mesh=mesh,
        out_type=jax.ShapeDtypeStruct((rows // 1024, 32, 32, _EMBED_DIM), jnp.float32),
        scratch_types=[
            pltpu.VMEM((b_per_w,), jnp.int32),
            pltpu.VMEM((b_per_w, _EMBED_DIM), jnp.float32),
            pltpu.SemaphoreType.DMA,
        ],
        compiler_params=pltpu.CompilerParams(use_tc_tiling_on_sc=False),
    )
    def gather_k(table_hbm, idx_hbm, out_hbm, idx_v, rows_v, sem):
        wid = lax.axis_index("s") * _NC + lax.axis_index("c")
        base = wid * b_per_w
        pltpu.sync_copy(idx_hbm.at[pl.ds(base, b_per_w)], idx_v)
        pltpu.async_copy(table_hbm.at[idx_v], rows_v, sem).wait()
        img = base // 1024
        i0 = (base % 1024) // 32
        handles = [
            pltpu.async_copy(
                rows_v.at[pl.ds(j * 32, 32)], out_hbm.at[img, i0 + j], sem)
            for j in range(b_per_w // 32)
        ]
        for h in handles:
            h.wait()

    return gather_k(table, idx)


def kernel(x, embeddings):
    in_shape = x.shape
    flat = x.reshape(-1, _EMBED_DIM)
    table = embeddings.T
    half = _ROWS // 2
    idx, l0 = _argmin_loss(flat, embeddings, _ROWS, 0)
    qtised = _sc_gather(table, idx, _ROWS)
    return qtised, l0[0, 0]


# speedup vs baseline: 1.2090x; 1.0100x over previous
import functools

import jax
import jax.numpy as jnp
from jax import lax
from jax.experimental import pallas as pl
from jax.experimental.pallas import tpu as pltpu
from jax.experimental.pallas import tpu_sc as plsc

_EMBED_DIM = 64
_EMBEDS = 1024
_ROWS = 16384
_BLK_R = 8192
_NB = _ROWS // _BLK_R

_NC = 2
_NS = 16
_NW = _NC * _NS
_B_PER_W = _ROWS // _NW


def _argmin_loss_body(flat_ref, emb_ref, idx_ref, loss_ref):
    i = pl.program_id(0)
    blk = flat_ref[...]
    emb = emb_ref[...]
    row_sq = jnp.sum(blk * blk, axis=1, keepdims=True)
    emb_sq = jnp.sum(emb * emb, axis=0, keepdims=True)
    half = _EMBEDS // 2
    m = None
    idxf = None
    for c in range(2):
        emb_c = emb[:, c * half:(c + 1) * half]
        prod_c = jnp.dot(blk, emb_c, preferred_element_type=jnp.float32)
        d_c = row_sq + emb_sq[:, c * half:(c + 1) * half] - 2.0 * prod_c
        m_c = jnp.min(d_c, axis=1, keepdims=True)
        iot = lax.broadcasted_iota(jnp.int32, d_c.shape, 1).astype(jnp.float32)
        i_c = jnp.min(jnp.where(d_c == m_c, iot, float(_EMBEDS)), axis=1)
        i_c = i_c + float(c * half)
        if m is None:
            m, idxf = m_c, i_c
        else:
            idxf = jnp.where(m_c[:, 0] < m[:, 0], i_c, idxf)
            m = jnp.minimum(m, m_c)
    idx_ref[...] = idxf.astype(jnp.int32)

    @pl.when(i == 0)
    def _init():
        loss_ref[0, 0] = 0.0

    loss_ref[0, 0] += jnp.sum(m)

    @pl.when(i == pl.num_programs(0) - 1)
    def _finish():
        loss_ref[0, 0] = loss_ref[0, 0] * (1.25 / (_ROWS * _EMBED_DIM))


def _argmin_loss(flat, embeddings, rows, off):
    nb = rows // _BLK_R
    off_nb = off // _BLK_R
    return pl.pallas_call(
        _argmin_loss_body,
        grid=(nb,),
        in_specs=[
            pl.BlockSpec((_BLK_R, _EMBED_DIM), lambda i: (i + off_nb, 0)),
            pl.BlockSpec((_EMBED_DIM, _EMBEDS), lambda i: (0, 0)),
        ],
        out_specs=[
            pl.BlockSpec((_BLK_R,), lambda i: (i,)),
            pl.BlockSpec(memory_space=pltpu.SMEM),
        ],
        out_shape=[
            jax.ShapeDtypeStruct((rows,), jnp.int32),
            jax.ShapeDtypeStruct((1, 1), jnp.float32),
        ],
        compiler_params=pltpu.CompilerParams(
            dimension_semantics=("arbitrary",),
        ),
    )(flat, embeddings)


def _sc_gather(table, idx, rows):
    b_per_w = rows // _NW
    mesh = plsc.VectorSubcoreMesh(
        core_axis_name="c", subcore_axis_name="s",
        num_cores=_NC, num_subcores=_NS,
    )

    @functools.partial(
        pl.kernel,
        mesh=mesh,
        out_type=jax.ShapeDtypeStruct((rows // 1024, 32, 32, _EMBED_DIM), jnp.float32),
        scratch_types=[
            pltpu.VMEM((b_per_w,), jnp.int32),
            pltpu.VMEM((b_per_w, _EMBED_DIM), jnp.float32),
            pltpu.SemaphoreType.DMA,
        ],
        compiler_params=pltpu.CompilerParams(use_tc_tiling_on_sc=False),
    )
    def gather_k(table_hbm, idx_hbm, out_hbm, idx_v, rows_v, sem):
        wid = lax.axis_index("s") * _NC + lax.axis_index("c")
        base = wid * b_per_w
        pltpu.sync_copy(idx_hbm.at[pl.ds(base, b_per_w)], idx_v)
        pltpu.async_copy(table_hbm.at[idx_v], rows_v, sem).wait()
        img = base // 1024
        i0 = (base % 1024) // 32
        handles = [
            pltpu.async_copy(
                rows_v.at[pl.ds(j * 32, 32)], out_hbm.at[img, i0 + j], sem)
            for j in range(b_per_w // 32)
        ]
        for h in handles:
            h.wait()

    return gather_k(table, idx)


def kernel(x, embeddings):
    in_shape = x.shape
    flat = x.reshape(-1, _EMBED_DIM)
    table = embeddings.T
    half = _ROWS // 2
    idx, l0 = _argmin_loss(flat, embeddings, _ROWS, 0)
    qtised = _sc_gather(table, idx, _ROWS)
    return qtised, l0[0, 0]
